# baseline (speedup 1.0000x reference)
import functools

import jax
import jax.numpy as jnp
from jax import lax
from jax.experimental import pallas as pl
from jax.experimental.pallas import tpu as pltpu
from jax.experimental.pallas import tpu_sc as plsc

E = 16
K = 2
D = 1024
H = 2048

R = 128
HT = 1024
NH = H // HT


def _gate_kernel(x_ref, gw_ref, idx_ref, sc_ref, imp_ref, load_ref, loss_ref):
    T = x_ref.shape[0]
    logits = jnp.dot(x_ref[...], gw_ref[...], preferred_element_type=jnp.float32)
    lane = lax.broadcasted_iota(jnp.int32, (T, E), 1)
    neg = jnp.float32(-3.0e38)

    m1 = jnp.max(logits, axis=1, keepdims=True)
    i1 = jnp.min(jnp.where(logits == m1, lane, E), axis=1, keepdims=True)
    hot1 = lane == i1
    l2 = jnp.where(hot1, neg, logits)
    m2 = jnp.max(l2, axis=1, keepdims=True)
    i2 = jnp.min(jnp.where(l2 == m2, lane, E), axis=1, keepdims=True)
    hot2 = lane == i2

    z = jnp.exp(m2 - m1)
    s1 = 1.0 / (1.0 + z)
    s2 = z / (1.0 + z)

    two = lax.broadcasted_iota(jnp.int32, (T, K), 1)
    idx_ref[...] = jnp.where(two == 0, i1, i2)
    sc_ref[...] = jnp.where(two == 0, s1, s2)

    sf = jnp.where(hot1, s1, 0.0) + jnp.where(hot2, s2, 0.0)
    imp = jnp.sum(sf, axis=0, keepdims=True)
    ld = jnp.sum(hot1.astype(jnp.float32) + hot2.astype(jnp.float32),
                 axis=0, keepdims=True)
    imp_ref[...] = imp
    load_ref[...] = ld

    def cv2(v):
        mu = jnp.mean(v, keepdims=True)
        var = jnp.mean((v - mu) ** 2, keepdims=True)
        return var / (mu * mu + 1e-10)

    loss_ref[...] = 0.01 * (cv2(imp) + cv2(ld))


def _run_gate(tok, gate_w):
    T = tok.shape[0]
    return pl.pallas_call(
        _gate_kernel,
        out_shape=[
            jax.ShapeDtypeStruct((T, K), jnp.int32),
            jax.ShapeDtypeStruct((T, K), jnp.float32),
            jax.ShapeDtypeStruct((1, E), jnp.float32),
            jax.ShapeDtypeStruct((1, E), jnp.float32),
            jax.ShapeDtypeStruct((1, 1), jnp.float32),
        ],
    )(tok, gate_w)


_NC = 2
_NS = 16
_NW = _NC * _NS
_CH = 64


def _sc_gather(table, idx):
    B = idx.shape[0]
    Dt = table.shape[1]
    b_per_w = B // _NW
    n_ch = b_per_w // _CH

    def body(tab_hbm, idx_hbm, out_hbm, idx_v, rows_v, sem):
        wid = lax.axis_index("s") * _NC + lax.axis_index("c")
        base = wid * b_per_w
        for c in range(n_ch):
            pltpu.sync_copy(idx_hbm.at[pl.ds(base + c * _CH, _CH)], idx_v)
            pltpu.async_copy(tab_hbm.at[idx_v], rows_v, sem).wait()
            pltpu.sync_copy(rows_v, out_hbm.at[pl.ds(base + c * _CH, _CH)])

    return pl.kernel(
        body,
        out_type=jax.ShapeDtypeStruct((B, Dt), jnp.float32),
        mesh=plsc.VectorSubcoreMesh(core_axis_name="c", subcore_axis_name="s"),
        scratch_types=[
            pltpu.VMEM((_CH,), jnp.int32),
            pltpu.VMEM((_CH, Dt), jnp.float32),
            pltpu.SemaphoreType.DMA,
        ],
    )(table, idx)


def _gmm_kernel(eid_ref, xs_ref, wg_ref, wu_ref, wd_ref, out_ref):
    xb = xs_ref[...]
    a = jnp.dot(xb, wg_ref[0], preferred_element_type=jnp.float32)
    b = jnp.dot(xb, wu_ref[0], preferred_element_type=jnp.float32)
    hh = a * jax.nn.sigmoid(a) * b
    out_ref[...] = jnp.dot(hh, wd_ref[0], preferred_element_type=jnp.float32)


def _run_gmm(x_sorted, w_gate, w_up, w_down, eid, grid_g):
    PAD = x_sorted.shape[0]
    grid_spec = pltpu.PrefetchScalarGridSpec(
        num_scalar_prefetch=1,
        grid=(grid_g,),
        in_specs=[
            pl.BlockSpec((R, D), lambda g, e: (g, 0)),
            pl.BlockSpec((1, D, H), lambda g, e: (e[g], 0, 0)),
            pl.BlockSpec((1, D, H), lambda g, e: (e[g], 0, 0)),
            pl.BlockSpec((1, H, D), lambda g, e: (e[g], 0, 0)),
        ],
        out_specs=pl.BlockSpec((R, D), lambda g, e: (g, 0)),
    )
    return pl.pallas_call(
        _gmm_kernel,
        grid_spec=grid_spec,
        out_shape=jax.ShapeDtypeStruct((PAD, D), jnp.float32),
    )(eid, x_sorted, w_gate, w_up, w_down)


def kernel(x, gate_w, w_gate_proj, w_up_proj, w_down_proj):
    orig_shape = x.shape
    tok = x.reshape(-1, D)
    T = tok.shape[0]
    TK = T * K
    NT = TK // R
    G = NT + E
    PAD = G * R

    idx, sc, imp2, load2, loss2 = _run_gate(tok, gate_w)

    e_flat = idx.reshape(-1)
    onehot = (e_flat[:, None] == jnp.arange(E, dtype=jnp.int32)[None, :])
    oh32 = onehot.astype(jnp.int32)
    csum = jnp.cumsum(oh32, axis=0)
    pos = csum - oh32
    counts = csum[-1]
    tiles_per_e = (counts + R - 1) // R
    cum_tiles = jnp.cumsum(tiles_per_e)
    pad_offs = (cum_tiles - tiles_per_e) * R
    offr = jnp.sum(oh32 * pad_offs[None, :], axis=1)
    posr = jnp.sum(oh32 * pos, axis=1)
    dest = offr + posr

    g_ids = jnp.arange(G, dtype=jnp.int32)
    eid = jnp.sum((cum_tiles[None, :] <= g_ids[:, None]).astype(jnp.int32),
                  axis=1)
    eid = jnp.minimum(eid, E - 1).astype(jnp.int32)

    perm = jnp.zeros((PAD,), jnp.int32).at[dest].set(
        jnp.arange(TK, dtype=jnp.int32))
    x_sorted = _sc_gather(tok, perm // K)

    out_sorted = _run_gmm(x_sorted, w_gate_proj, w_up_proj, w_down_proj,
                          eid, G)

    out_pair = _sc_gather(out_sorted, dest).reshape(T, K, D)
    y = jnp.sum(out_pair * sc[:, :, None], axis=1)

    hidden = y.reshape(orig_shape)
    balance_loss = loss2[0, 0]
    num_dropped = jnp.array(0, dtype=jnp.int32)
    return hidden, balance_loss, num_dropped, load2[0], imp2[0]
